# baseline (speedup 1.0000x reference)
import functools

import jax
import jax.numpy as jnp
from jax import lax
from jax.experimental import pallas as pl
from jax.experimental.pallas import tpu as pltpu
from jax.experimental.pallas import tpu_sc as plsc

N_NODES = 10000
N_EDGES = 320000
IN_CH = 128
HID_CH = 64
OUT_CH = 64

NC, NS = 2, 16
NW = NC * NS
CHUNK = 128
NBUF = 2
NA, NB = 78, 79
NBMAX = max(NA, NB)
TOTCH = NS * (NA + NB)
EPAD = TOTCH * CHUNK
NPAD = 10112
ROWS_PER_SUB = NPAD // NS
BLK = NPAD // 8

@functools.cache
def _sc_aggregate_fn():
    mesh = plsc.VectorSubcoreMesh(core_axis_name="c", subcore_axis_name="s",
                                  num_cores=NC, num_subcores=NS)
    return functools.partial(
        pl.kernel,
        out_type=jax.ShapeDtypeStruct((NC, NPAD, HID_CH), jnp.float32),
        mesh=mesh,
        scratch_types=[
            pltpu.VMEM((NBMAX, CHUNK), jnp.int32),
            pltpu.VMEM((NBMAX, CHUNK), jnp.int32),
            pltpu.VMEM((NBUF, CHUNK, HID_CH), jnp.float32),
            pltpu.VMEM_SHARED((NPAD, HID_CH), jnp.float32),
            pltpu.VMEM_SHARED((NPAD, HID_CH), jnp.float32),
            pltpu.SemaphoreType.DMA,
            pltpu.SemaphoreType.DMA,
        ],
        compiler_params=pltpu.CompilerParams(use_tc_tiling_on_sc=False),
    )(_sc_aggregate_body)


def _sc_aggregate_body(h_hbm, srca_hbm, dsta_hbm, srcb_hbm, dstb_hbm, z_hbm,
                       out_hbm, src_v, dst_v, rows_v, acc, h_s, sem0, sem1):
    sems = (sem0, sem1)
    c = lax.axis_index("c")
    s = lax.axis_index("s")
    row0 = s * ROWS_PER_SUB
    pltpu.sync_copy(z_hbm, acc.at[pl.ds(row0, ROWS_PER_SUB)])
    pltpu.sync_copy(h_hbm.at[pl.ds(row0, ROWS_PER_SUB)],
                    h_s.at[pl.ds(row0, ROWS_PER_SUB)])

    def run(src_hbm, dst_hbm, n):
        pltpu.sync_copy(src_hbm.at[s], src_v.at[pl.ds(0, n)])
        pltpu.sync_copy(dst_hbm.at[s], dst_v.at[pl.ds(0, n)])
        plsc.subcore_barrier()

        pltpu.async_copy(h_hbm.at[src_v.at[0]], rows_v.at[0], sems[0])

        def body(j, _):
            buf = lax.rem(j, 2)

            def step(h_cur, h_nxt, cur_sem, nxt_sem, cur_buf, nxt_buf):
                pltpu.make_async_copy(h_cur.at[src_v.at[j]],
                                      rows_v.at[cur_buf], cur_sem).wait()
                @pl.when(j + 1 < n)
                def _():
                    pltpu.async_copy(h_nxt.at[src_v.at[j + 1]],
                                     rows_v.at[nxt_buf], nxt_sem)
                pltpu.sync_copy(rows_v.at[cur_buf], acc.at[dst_v.at[j]],
                                add=True)

            @pl.when(buf == 0)
            def _():
                step(h_hbm, h_s, sems[0], sems[1], 0, 1)

            @pl.when(buf == 1)
            def _():
                step(h_s, h_hbm, sems[1], sems[0], 1, 0)

            return 0

        lax.fori_loop(0, n, body, 0)

    @pl.when(c == 0)
    def _():
        run(srca_hbm, dsta_hbm, NA)

    @pl.when(c == 1)
    def _():
        run(srcb_hbm, dstb_hbm, NB)

    plsc.subcore_barrier()
    pltpu.sync_copy(acc.at[pl.ds(row0, ROWS_PER_SUB)],
                    out_hbm.at[c, pl.ds(row0, ROWS_PER_SUB)])


def _mm1_body(x_ref, w_ref, o_ref):
    o_ref[...] = jnp.dot(x_ref[...], w_ref[...],
                         preferred_element_type=jnp.float32)


def _mid_body(p_ref, b_ref, w_ref, o_ref):
    i = pl.program_id(0)
    h = p_ref[0] + p_ref[1] + b_ref[...]
    h = jnp.maximum(h, 0.0)
    rows = i * BLK + lax.broadcasted_iota(jnp.int32, (BLK, 1), 0)
    h = jnp.where(rows < N_NODES, h, 0.0)
    o_ref[...] = jnp.dot(h, w_ref[...], preferred_element_type=jnp.float32)


def _out_body(p_ref, b_ref, o_ref):
    z = p_ref[0] + p_ref[1] + b_ref[...]
    m = jnp.max(z, axis=-1, keepdims=True)
    e = jnp.exp(z - m)
    o_ref[...] = (z - m) - jnp.log(jnp.sum(e, axis=-1, keepdims=True))


def kernel(x, edge_index, W1, b1, W2, b2):
    x_pad = jnp.pad(x, ((0, NPAD - N_NODES), (0, 0)))
    ei = edge_index.astype(jnp.int32)
    src = jnp.pad(ei[0], (0, EPAD - N_EDGES), constant_values=N_NODES)
    dst = jnp.pad(ei[1], (0, EPAD - N_EDGES), constant_values=0)
    srcc = src.reshape(TOTCH, CHUNK)
    dstc = dst.reshape(TOTCH, CHUNK)
    srcA = srcc[:NS * NA].reshape(NS, NA, CHUNK)
    dstA = dstc[:NS * NA].reshape(NS, NA, CHUNK)
    srcB = srcc[NS * NA:].reshape(NS, NB, CHUNK)
    dstB = dstc[NS * NA:].reshape(NS, NB, CHUNK)
    zblk = jnp.zeros((ROWS_PER_SUB, HID_CH), jnp.float32)

    h1 = pl.pallas_call(
        _mm1_body,
        grid=(8,),
        in_specs=[pl.BlockSpec((BLK, IN_CH), lambda i: (i, 0)),
                  pl.BlockSpec((IN_CH, HID_CH), lambda i: (0, 0))],
        out_specs=pl.BlockSpec((BLK, HID_CH), lambda i: (i, 0)),
        out_shape=jax.ShapeDtypeStruct((NPAD, HID_CH), jnp.float32),
    )(x_pad, W1)

    p1 = _sc_aggregate_fn()(h1, srcA, dstA, srcB, dstB, zblk)

    h2 = pl.pallas_call(
        _mid_body,
        grid=(8,),
        in_specs=[pl.BlockSpec((NC, BLK, HID_CH), lambda i: (0, i, 0)),
                  pl.BlockSpec((1, HID_CH), lambda i: (0, 0)),
                  pl.BlockSpec((HID_CH, OUT_CH), lambda i: (0, 0))],
        out_specs=pl.BlockSpec((BLK, OUT_CH), lambda i: (i, 0)),
        out_shape=jax.ShapeDtypeStruct((NPAD, OUT_CH), jnp.float32),
    )(p1, b1.reshape(1, HID_CH), W2)

    p2 = _sc_aggregate_fn()(h2, srcA, dstA, srcB, dstB, zblk)

    out = pl.pallas_call(
        _out_body,
        grid=(8,),
        in_specs=[pl.BlockSpec((NC, BLK, OUT_CH), lambda i: (0, i, 0)),
                  pl.BlockSpec((1, OUT_CH), lambda i: (0, 0))],
        out_specs=pl.BlockSpec((BLK, OUT_CH), lambda i: (i, 0)),
        out_shape=jax.ShapeDtypeStruct((NPAD, OUT_CH), jnp.float32),
    )(p2, b2.reshape(1, OUT_CH))

    return out[:N_NODES]


# speedup vs baseline: 1.1675x; 1.1675x over previous
import functools

import jax
import jax.numpy as jnp
from jax import lax
from jax.experimental import pallas as pl
from jax.experimental.pallas import tpu as pltpu
from jax.experimental.pallas import tpu_sc as plsc

N_NODES = 10000
N_EDGES = 320000
IN_CH = 128
HID_CH = 64
OUT_CH = 64

NC, NS = 2, 16
NW = NC * NS
CHUNK = 128
NBUF = 2
NA, NB = 78, 79
NBMAX = max(NA, NB)
TOTCH = NS * (NA + NB)
EPAD = TOTCH * CHUNK
NPAD = 10112
ROWS_PER_SUB = NPAD // NS
BLK = NPAD // 8

@functools.cache
def _sc_aggregate_fn():
    mesh = plsc.VectorSubcoreMesh(core_axis_name="c", subcore_axis_name="s",
                                  num_cores=NC, num_subcores=NS)
    return functools.partial(
        pl.kernel,
        out_type=jax.ShapeDtypeStruct((NC, NPAD, HID_CH), jnp.float32),
        mesh=mesh,
        scratch_types=[
            pltpu.VMEM((NBMAX, CHUNK), jnp.int32),
            pltpu.VMEM((NBMAX, CHUNK), jnp.int32),
            pltpu.VMEM((NBUF, CHUNK, HID_CH), jnp.float32),
            pltpu.VMEM_SHARED((NPAD, HID_CH), jnp.float32),
            pltpu.VMEM_SHARED((NPAD, HID_CH), jnp.float32),
            pltpu.SemaphoreType.DMA,
            pltpu.SemaphoreType.DMA,
            pltpu.SemaphoreType.DMA,
            pltpu.SemaphoreType.DMA,
        ],
        compiler_params=pltpu.CompilerParams(use_tc_tiling_on_sc=False),
    )(_sc_aggregate_body)


def _sc_aggregate_body(h_hbm, srca_hbm, dsta_hbm, srcb_hbm, dstb_hbm, z_hbm,
                       out_hbm, src_v, dst_v, rows_v, acc, h_s,
                       sem0, sem1, sem2, sem3):
    sem_g = (sem0, sem1)
    sem_s = (sem2, sem3)
    c = lax.axis_index("c")
    s = lax.axis_index("s")
    row0 = s * ROWS_PER_SUB
    pltpu.sync_copy(z_hbm, acc.at[pl.ds(row0, ROWS_PER_SUB)])
    pltpu.sync_copy(h_hbm.at[pl.ds(row0, ROWS_PER_SUB)],
                    h_s.at[pl.ds(row0, ROWS_PER_SUB)])

    def run(src_hbm, dst_hbm, n):
        pltpu.sync_copy(src_hbm.at[s], src_v.at[pl.ds(0, n)])
        pltpu.sync_copy(dst_hbm.at[s], dst_v.at[pl.ds(0, n)])
        plsc.subcore_barrier()

        pltpu.async_copy(h_s.at[src_v.at[0]], rows_v.at[0], sem_g[0])

        def body(j, _):
            buf = lax.rem(j, 2)

            def step(cur, nxt):
                pltpu.make_async_copy(h_s.at[src_v.at[j]],
                                      rows_v.at[cur], sem_g[cur]).wait()
                pltpu.async_copy(rows_v.at[cur], acc.at[dst_v.at[j]],
                                 sem_s[cur], add=True)

                @pl.when(j >= 1)
                def _():
                    pltpu.make_async_copy(rows_v.at[nxt],
                                          acc.at[dst_v.at[0]],
                                          sem_s[nxt]).wait()

                @pl.when(j + 1 < n)
                def _():
                    pltpu.async_copy(h_s.at[src_v.at[j + 1]],
                                     rows_v.at[nxt], sem_g[nxt])

            @pl.when(buf == 0)
            def _():
                step(0, 1)

            @pl.when(buf == 1)
            def _():
                step(1, 0)

            return 0

        lax.fori_loop(0, n, body, 0)
        pltpu.make_async_copy(rows_v.at[(n - 1) % 2], acc.at[dst_v.at[0]],
                              sem_s[(n - 1) % 2]).wait()

    @pl.when(c == 0)
    def _():
        run(srca_hbm, dsta_hbm, NA)

    @pl.when(c == 1)
    def _():
        run(srcb_hbm, dstb_hbm, NB)

    plsc.subcore_barrier()
    pltpu.sync_copy(acc.at[pl.ds(row0, ROWS_PER_SUB)],
                    out_hbm.at[c, pl.ds(row0, ROWS_PER_SUB)])


def _mm1_body(x_ref, w_ref, o_ref):
    o_ref[...] = jnp.dot(x_ref[...], w_ref[...],
                         preferred_element_type=jnp.float32)


def _mid_body(p_ref, b_ref, w_ref, o_ref):
    i = pl.program_id(0)
    h = p_ref[0] + p_ref[1] + b_ref[...]
    h = jnp.maximum(h, 0.0)
    rows = i * BLK + lax.broadcasted_iota(jnp.int32, (BLK, 1), 0)
    h = jnp.where(rows < N_NODES, h, 0.0)
    o_ref[...] = jnp.dot(h, w_ref[...], preferred_element_type=jnp.float32)


def _out_body(p_ref, b_ref, o_ref):
    z = p_ref[0] + p_ref[1] + b_ref[...]
    m = jnp.max(z, axis=-1, keepdims=True)
    e = jnp.exp(z - m)
    o_ref[...] = (z - m) - jnp.log(jnp.sum(e, axis=-1, keepdims=True))


def kernel(x, edge_index, W1, b1, W2, b2):
    x_pad = jnp.pad(x, ((0, NPAD - N_NODES), (0, 0)))
    ei = edge_index.astype(jnp.int32)
    src = jnp.pad(ei[0], (0, EPAD - N_EDGES), constant_values=N_NODES)
    dst = jnp.pad(ei[1], (0, EPAD - N_EDGES), constant_values=0)
    srcc = src.reshape(TOTCH, CHUNK)
    dstc = dst.reshape(TOTCH, CHUNK)
    srcA = srcc[:NS * NA].reshape(NS, NA, CHUNK)
    dstA = dstc[:NS * NA].reshape(NS, NA, CHUNK)
    srcB = srcc[NS * NA:].reshape(NS, NB, CHUNK)
    dstB = dstc[NS * NA:].reshape(NS, NB, CHUNK)
    zblk = jnp.zeros((ROWS_PER_SUB, HID_CH), jnp.float32)

    h1 = pl.pallas_call(
        _mm1_body,
        grid=(8,),
        in_specs=[pl.BlockSpec((BLK, IN_CH), lambda i: (i, 0)),
                  pl.BlockSpec((IN_CH, HID_CH), lambda i: (0, 0))],
        out_specs=pl.BlockSpec((BLK, HID_CH), lambda i: (i, 0)),
        out_shape=jax.ShapeDtypeStruct((NPAD, HID_CH), jnp.float32),
    )(x_pad, W1)

    p1 = _sc_aggregate_fn()(h1, srcA, dstA, srcB, dstB, zblk)

    h2 = pl.pallas_call(
        _mid_body,
        grid=(8,),
        in_specs=[pl.BlockSpec((NC, BLK, HID_CH), lambda i: (0, i, 0)),
                  pl.BlockSpec((1, HID_CH), lambda i: (0, 0)),
                  pl.BlockSpec((HID_CH, OUT_CH), lambda i: (0, 0))],
        out_specs=pl.BlockSpec((BLK, OUT_CH), lambda i: (i, 0)),
        out_shape=jax.ShapeDtypeStruct((NPAD, OUT_CH), jnp.float32),
    )(p1, b1.reshape(1, HID_CH), W2)

    p2 = _sc_aggregate_fn()(h2, srcA, dstA, srcB, dstB, zblk)

    out = pl.pallas_call(
        _out_body,
        grid=(8,),
        in_specs=[pl.BlockSpec((NC, BLK, OUT_CH), lambda i: (0, i, 0)),
                  pl.BlockSpec((1, OUT_CH), lambda i: (0, 0))],
        out_specs=pl.BlockSpec((BLK, OUT_CH), lambda i: (i, 0)),
        out_shape=jax.ShapeDtypeStruct((NPAD, OUT_CH), jnp.float32),
    )(p2, b2.reshape(1, OUT_CH))

    return out[:N_NODES]


# speedup vs baseline: 1.2727x; 1.0901x over previous
import functools

import jax
import jax.numpy as jnp
from jax import lax
from jax.experimental import pallas as pl
from jax.experimental.pallas import tpu as pltpu
from jax.experimental.pallas import tpu_sc as plsc

N_NODES = 10000
N_EDGES = 320000
IN_CH = 128
HID_CH = 64
OUT_CH = 64

NC, NS = 2, 16
NW = NC * NS
CHUNK = 128
NBUF = 3
NA, NB = 78, 79
NBMAX = max(NA, NB)
TOTCH = NS * (NA + NB)
EPAD = TOTCH * CHUNK
NPAD = 10112
ROWS_PER_SUB = NPAD // NS
BLK = NPAD // 8

@functools.cache
def _sc_aggregate_fn():
    mesh = plsc.VectorSubcoreMesh(core_axis_name="c", subcore_axis_name="s",
                                  num_cores=NC, num_subcores=NS)
    return functools.partial(
        pl.kernel,
        out_type=jax.ShapeDtypeStruct((NC, NPAD, HID_CH), jnp.float32),
        mesh=mesh,
        scratch_types=[
            pltpu.VMEM((NBMAX, CHUNK), jnp.int32),
            pltpu.VMEM((NBMAX, CHUNK), jnp.int32),
            pltpu.VMEM((NBUF, CHUNK, HID_CH), jnp.float32),
            pltpu.VMEM_SHARED((NPAD, HID_CH), jnp.float32),
            pltpu.VMEM_SHARED((NPAD, HID_CH), jnp.float32),
            pltpu.SemaphoreType.DMA,
            pltpu.SemaphoreType.DMA,
            pltpu.SemaphoreType.DMA,
            pltpu.SemaphoreType.DMA,
            pltpu.SemaphoreType.DMA,
            pltpu.SemaphoreType.DMA,
        ],
        compiler_params=pltpu.CompilerParams(use_tc_tiling_on_sc=False),
    )(_sc_aggregate_body)


def _sc_aggregate_body(h_hbm, srca_hbm, dsta_hbm, srcb_hbm, dstb_hbm, z_hbm,
                       out_hbm, src_v, dst_v, rows_v, acc, h_s,
                       sem0, sem1, sem2, sem3, sem4, sem5):
    sem_g = (sem0, sem1, sem2)
    sem_s = (sem3, sem4, sem5)
    c = lax.axis_index("c")
    s = lax.axis_index("s")
    row0 = s * ROWS_PER_SUB
    pltpu.sync_copy(z_hbm, acc.at[pl.ds(row0, ROWS_PER_SUB)])
    pltpu.sync_copy(h_hbm.at[pl.ds(row0, ROWS_PER_SUB)],
                    h_s.at[pl.ds(row0, ROWS_PER_SUB)])

    def run(src_hbm, dst_hbm, n):
        pltpu.sync_copy(src_hbm.at[s], src_v.at[pl.ds(0, n)])
        pltpu.sync_copy(dst_hbm.at[s], dst_v.at[pl.ds(0, n)])
        plsc.subcore_barrier()

        pltpu.async_copy(h_s.at[src_v.at[0]], rows_v.at[0], sem_g[0])
        pltpu.async_copy(h_s.at[src_v.at[1]], rows_v.at[1], sem_g[1])

        def body(j, _):
            buf = lax.rem(j, 3)

            def step(cur, nn):
                pltpu.make_async_copy(h_s.at[src_v.at[j]],
                                      rows_v.at[cur], sem_g[cur]).wait()
                pltpu.async_copy(rows_v.at[cur], acc.at[dst_v.at[j]],
                                 sem_s[cur], add=True)

                @pl.when(j >= 1)
                def _():
                    pltpu.make_async_copy(rows_v.at[nn],
                                          acc.at[dst_v.at[0]],
                                          sem_s[nn]).wait()

                @pl.when(j + 2 < n)
                def _():
                    pltpu.async_copy(h_s.at[src_v.at[j + 2]],
                                     rows_v.at[nn], sem_g[nn])

            for k in range(3):
                @pl.when(buf == k)
                def _(k=k):
                    step(k, (k + 2) % 3)

            return 0

        lax.fori_loop(0, n, body, 0)
        pltpu.make_async_copy(rows_v.at[(n - 1) % 3], acc.at[dst_v.at[0]],
                              sem_s[(n - 1) % 3]).wait()

    @pl.when(c == 0)
    def _():
        run(srca_hbm, dsta_hbm, NA)

    @pl.when(c == 1)
    def _():
        run(srcb_hbm, dstb_hbm, NB)

    plsc.subcore_barrier()
    pltpu.sync_copy(acc.at[pl.ds(row0, ROWS_PER_SUB)],
                    out_hbm.at[c, pl.ds(row0, ROWS_PER_SUB)])


def _mm1_body(x_ref, w_ref, o_ref):
    o_ref[...] = jnp.dot(x_ref[...], w_ref[...],
                         preferred_element_type=jnp.float32)


def _mid_body(p_ref, b_ref, w_ref, o_ref):
    i = pl.program_id(0)
    h = p_ref[0] + p_ref[1] + b_ref[...]
    h = jnp.maximum(h, 0.0)
    rows = i * BLK + lax.broadcasted_iota(jnp.int32, (BLK, 1), 0)
    h = jnp.where(rows < N_NODES, h, 0.0)
    o_ref[...] = jnp.dot(h, w_ref[...], preferred_element_type=jnp.float32)


def _out_body(p_ref, b_ref, o_ref):
    z = p_ref[0] + p_ref[1] + b_ref[...]
    m = jnp.max(z, axis=-1, keepdims=True)
    e = jnp.exp(z - m)
    o_ref[...] = (z - m) - jnp.log(jnp.sum(e, axis=-1, keepdims=True))


def kernel(x, edge_index, W1, b1, W2, b2):
    x_pad = jnp.pad(x, ((0, NPAD - N_NODES), (0, 0)))
    ei = edge_index.astype(jnp.int32)
    src = jnp.pad(ei[0], (0, EPAD - N_EDGES), constant_values=N_NODES)
    dst = jnp.pad(ei[1], (0, EPAD - N_EDGES), constant_values=0)
    srcc = src.reshape(TOTCH, CHUNK)
    dstc = dst.reshape(TOTCH, CHUNK)
    srcA = srcc[:NS * NA].reshape(NS, NA, CHUNK)
    dstA = dstc[:NS * NA].reshape(NS, NA, CHUNK)
    srcB = srcc[NS * NA:].reshape(NS, NB, CHUNK)
    dstB = dstc[NS * NA:].reshape(NS, NB, CHUNK)
    zblk = jnp.zeros((ROWS_PER_SUB, HID_CH), jnp.float32)

    h1 = pl.pallas_call(
        _mm1_body,
        grid=(8,),
        in_specs=[pl.BlockSpec((BLK, IN_CH), lambda i: (i, 0)),
                  pl.BlockSpec((IN_CH, HID_CH), lambda i: (0, 0))],
        out_specs=pl.BlockSpec((BLK, HID_CH), lambda i: (i, 0)),
        out_shape=jax.ShapeDtypeStruct((NPAD, HID_CH), jnp.float32),
    )(x_pad, W1)

    p1 = _sc_aggregate_fn()(h1, srcA, dstA, srcB, dstB, zblk)

    h2 = pl.pallas_call(
        _mid_body,
        grid=(8,),
        in_specs=[pl.BlockSpec((NC, BLK, HID_CH), lambda i: (0, i, 0)),
                  pl.BlockSpec((1, HID_CH), lambda i: (0, 0)),
                  pl.BlockSpec((HID_CH, OUT_CH), lambda i: (0, 0))],
        out_specs=pl.BlockSpec((BLK, OUT_CH), lambda i: (i, 0)),
        out_shape=jax.ShapeDtypeStruct((NPAD, OUT_CH), jnp.float32),
    )(p1, b1.reshape(1, HID_CH), W2)

    p2 = _sc_aggregate_fn()(h2, srcA, dstA, srcB, dstB, zblk)

    out = pl.pallas_call(
        _out_body,
        grid=(8,),
        in_specs=[pl.BlockSpec((NC, BLK, OUT_CH), lambda i: (0, i, 0)),
                  pl.BlockSpec((1, OUT_CH), lambda i: (0, 0))],
        out_specs=pl.BlockSpec((BLK, OUT_CH), lambda i: (i, 0)),
        out_shape=jax.ShapeDtypeStruct((NPAD, OUT_CH), jnp.float32),
    )(p2, b2.reshape(1, OUT_CH))

    return out[:N_NODES]


# speedup vs baseline: 1.2937x; 1.0165x over previous
import functools

import jax
import jax.numpy as jnp
from jax import lax
from jax.experimental import pallas as pl
from jax.experimental.pallas import tpu as pltpu
from jax.experimental.pallas import tpu_sc as plsc

N_NODES = 10000
N_EDGES = 320000
IN_CH = 128
HID_CH = 64
OUT_CH = 64

NC, NS = 2, 16
NW = NC * NS
CHUNK = 128
NBUF = 3
NA, NB = 78, 79
NBMAX = max(NA, NB)
TOTCH = NS * (NA + NB)
EPAD = TOTCH * CHUNK
NPAD = 10112
ROWS_PER_SUB = NPAD // NS
BLK = NPAD // 8

@functools.cache
def _sc_aggregate_fn():
    mesh = plsc.VectorSubcoreMesh(core_axis_name="c", subcore_axis_name="s",
                                  num_cores=NC, num_subcores=NS)
    return functools.partial(
        pl.kernel,
        out_type=jax.ShapeDtypeStruct((NC, NPAD, HID_CH), jnp.float32),
        mesh=mesh,
        scratch_types=[
            pltpu.VMEM((NBMAX, CHUNK), jnp.int32),
            pltpu.VMEM((NBMAX, CHUNK), jnp.int32),
            pltpu.VMEM((NBUF, CHUNK, HID_CH), jnp.float32),
            pltpu.VMEM_SHARED((NPAD, HID_CH), jnp.float32),
            pltpu.VMEM_SHARED((NPAD, HID_CH), jnp.float32),
            pltpu.SemaphoreType.DMA,
            pltpu.SemaphoreType.DMA,
            pltpu.SemaphoreType.DMA,
            pltpu.SemaphoreType.DMA,
            pltpu.SemaphoreType.DMA,
            pltpu.SemaphoreType.DMA,
        ],
        compiler_params=pltpu.CompilerParams(use_tc_tiling_on_sc=False),
    )(_sc_aggregate_body)


def _sc_aggregate_body(h_hbm, srca_hbm, dsta_hbm, srcb_hbm, dstb_hbm, z_hbm,
                       out_hbm, src_v, dst_v, rows_v, acc, h_s,
                       sem0, sem1, sem2, sem3, sem4, sem5):
    sem_g = (sem0, sem1, sem2)
    sem_s = (sem3, sem4, sem5)
    c = lax.axis_index("c")
    s = lax.axis_index("s")
    row0 = s * ROWS_PER_SUB
    def run(src_hbm, dst_hbm, n):
        pltpu.async_copy(z_hbm, acc.at[pl.ds(row0, ROWS_PER_SUB)], sem_s[0])
        pltpu.async_copy(h_hbm.at[pl.ds(row0, ROWS_PER_SUB)],
                         h_s.at[pl.ds(row0, ROWS_PER_SUB)], sem_s[1])
        pltpu.async_copy(src_hbm.at[s], src_v.at[pl.ds(0, n)], sem_g[0])
        pltpu.async_copy(dst_hbm.at[s], dst_v.at[pl.ds(0, n)], sem_g[1])
        pltpu.make_async_copy(z_hbm, acc.at[pl.ds(row0, ROWS_PER_SUB)],
                              sem_s[0]).wait()
        pltpu.make_async_copy(h_hbm.at[pl.ds(row0, ROWS_PER_SUB)],
                              h_s.at[pl.ds(row0, ROWS_PER_SUB)],
                              sem_s[1]).wait()
        pltpu.make_async_copy(src_hbm.at[s], src_v.at[pl.ds(0, n)],
                              sem_g[0]).wait()
        pltpu.make_async_copy(dst_hbm.at[s], dst_v.at[pl.ds(0, n)],
                              sem_g[1]).wait()
        plsc.subcore_barrier()

        pltpu.async_copy(h_s.at[src_v.at[0]], rows_v.at[0], sem_g[0])
        pltpu.async_copy(h_s.at[src_v.at[1]], rows_v.at[1], sem_g[1])

        def body(j, _):
            buf = lax.rem(j, 3)

            def step(cur, nn):
                pltpu.make_async_copy(h_s.at[src_v.at[j]],
                                      rows_v.at[cur], sem_g[cur]).wait()
                pltpu.async_copy(rows_v.at[cur], acc.at[dst_v.at[j]],
                                 sem_s[cur], add=True)

                @pl.when(j >= 1)
                def _():
                    pltpu.make_async_copy(rows_v.at[nn],
                                          acc.at[dst_v.at[0]],
                                          sem_s[nn]).wait()

                @pl.when(j + 2 < n)
                def _():
                    pltpu.async_copy(h_s.at[src_v.at[j + 2]],
                                     rows_v.at[nn], sem_g[nn])

            for k in range(3):
                @pl.when(buf == k)
                def _(k=k):
                    step(k, (k + 2) % 3)

            return 0

        lax.fori_loop(0, n, body, 0)
        pltpu.make_async_copy(rows_v.at[(n - 1) % 3], acc.at[dst_v.at[0]],
                              sem_s[(n - 1) % 3]).wait()

    @pl.when(c == 0)
    def _():
        run(srca_hbm, dsta_hbm, NA)

    @pl.when(c == 1)
    def _():
        run(srcb_hbm, dstb_hbm, NB)

    plsc.subcore_barrier()
    pltpu.sync_copy(acc.at[pl.ds(row0, ROWS_PER_SUB)],
                    out_hbm.at[c, pl.ds(row0, ROWS_PER_SUB)])


def _mm1_body(x_ref, w_ref, o_ref):
    o_ref[...] = jnp.dot(x_ref[...], w_ref[...],
                         preferred_element_type=jnp.float32)


def _mid_body(p_ref, b_ref, w_ref, o_ref):
    i = pl.program_id(0)
    h = p_ref[0] + p_ref[1] + b_ref[...]
    h = jnp.maximum(h, 0.0)
    rows = i * BLK + lax.broadcasted_iota(jnp.int32, (BLK, 1), 0)
    h = jnp.where(rows < N_NODES, h, 0.0)
    o_ref[...] = jnp.dot(h, w_ref[...], preferred_element_type=jnp.float32)


def _out_body(p_ref, b_ref, o_ref):
    z = p_ref[0] + p_ref[1] + b_ref[...]
    m = jnp.max(z, axis=-1, keepdims=True)
    e = jnp.exp(z - m)
    o_ref[...] = (z - m) - jnp.log(jnp.sum(e, axis=-1, keepdims=True))


def kernel(x, edge_index, W1, b1, W2, b2):
    x_pad = jnp.pad(x, ((0, NPAD - N_NODES), (0, 0)))
    ei = edge_index.astype(jnp.int32)
    src = jnp.pad(ei[0], (0, EPAD - N_EDGES), constant_values=N_NODES)
    dst = jnp.pad(ei[1], (0, EPAD - N_EDGES), constant_values=0)
    srcc = src.reshape(TOTCH, CHUNK)
    dstc = dst.reshape(TOTCH, CHUNK)
    srcA = srcc[:NS * NA].reshape(NS, NA, CHUNK)
    dstA = dstc[:NS * NA].reshape(NS, NA, CHUNK)
    srcB = srcc[NS * NA:].reshape(NS, NB, CHUNK)
    dstB = dstc[NS * NA:].reshape(NS, NB, CHUNK)
    zblk = jnp.zeros((ROWS_PER_SUB, HID_CH), jnp.float32)

    h1 = pl.pallas_call(
        _mm1_body,
        grid=(8,),
        in_specs=[pl.BlockSpec((BLK, IN_CH), lambda i: (i, 0)),
                  pl.BlockSpec((IN_CH, HID_CH), lambda i: (0, 0))],
        out_specs=pl.BlockSpec((BLK, HID_CH), lambda i: (i, 0)),
        out_shape=jax.ShapeDtypeStruct((NPAD, HID_CH), jnp.float32),
    )(x_pad, W1)

    p1 = _sc_aggregate_fn()(h1, srcA, dstA, srcB, dstB, zblk)

    h2 = pl.pallas_call(
        _mid_body,
        grid=(8,),
        in_specs=[pl.BlockSpec((NC, BLK, HID_CH), lambda i: (0, i, 0)),
                  pl.BlockSpec((1, HID_CH), lambda i: (0, 0)),
                  pl.BlockSpec((HID_CH, OUT_CH), lambda i: (0, 0))],
        out_specs=pl.BlockSpec((BLK, OUT_CH), lambda i: (i, 0)),
        out_shape=jax.ShapeDtypeStruct((NPAD, OUT_CH), jnp.float32),
    )(p1, b1.reshape(1, HID_CH), W2)

    p2 = _sc_aggregate_fn()(h2, srcA, dstA, srcB, dstB, zblk)

    out = pl.pallas_call(
        _out_body,
        grid=(8,),
        in_specs=[pl.BlockSpec((NC, BLK, OUT_CH), lambda i: (0, i, 0)),
                  pl.BlockSpec((1, OUT_CH), lambda i: (0, 0))],
        out_specs=pl.BlockSpec((BLK, OUT_CH), lambda i: (i, 0)),
        out_shape=jax.ShapeDtypeStruct((NPAD, OUT_CH), jnp.float32),
    )(p2, b2.reshape(1, OUT_CH))

    return out[:N_NODES]
